# baseline (speedup 1.0000x reference)
import jax
import jax.numpy as jnp
from jax import lax
from jax.experimental import pallas as pl
from jax.experimental.pallas import tpu as pltpu

B, N, D_IN, D_OUT = 16, 4096, 64, 64
SEG = 8
ROWS = N // SEG
NSLOT = 4


def _body(nu_ref, g_ref, feat_hbm, v_ref, b_ref, out_hbm,
          inbuf, outbuf, fsem, osem):
    def in_copy(b, slot, s):
        return pltpu.make_async_copy(
            feat_hbm.at[b, pl.ds(s * ROWS, ROWS), :],
            inbuf.at[slot, pl.ds(s * ROWS, ROWS), :],
            fsem.at[slot, s])

    def out_copy(b, slot, s):
        return pltpu.make_async_copy(
            outbuf.at[slot, pl.ds(s * ROWS, ROWS), :],
            out_hbm.at[b, pl.ds(s * ROWS, ROWS), :],
            osem.at[slot, s])

    v = v_ref[...]
    norm = jnp.sqrt(jnp.sum(v * v))
    w = v * (g_ref[0] / norm)
    bias = b_ref[...]
    row_ids = lax.broadcasted_iota(jnp.int32, (N, D_IN), 0)

    for b in range(min(NSLOT - 1, B)):
        for s in range(SEG):
            in_copy(b, b % NSLOT, s).start()

    for b in range(B):
        slot = b % NSLOT
        nxt = b + NSLOT - 1
        if nxt < B:
            for s in range(SEG):
                in_copy(nxt, nxt % NSLOT, s).start()
        for s in range(SEG):
            in_copy(b, slot, s).wait()
        if b >= NSLOT:
            for s in range(SEG):
                out_copy(b - NSLOT, slot, s).wait()
        x = inbuf[slot]
        nu = nu_ref[b]
        masked = jnp.where(row_ids < nu, x, -jnp.inf)
        fmax = jnp.max(masked, axis=0, keepdims=True)
        h = x - fmax
        out = lax.dot_general(h, w, (((1,), (1,)), ((), ())),
                              preferred_element_type=jnp.float32)
        outbuf[slot] = jnp.maximum(out + bias, 0.0)
        for s in range(SEG):
            out_copy(b, slot, s).start()

    for b in range(max(B - NSLOT, 0), B):
        for s in range(SEG):
            out_copy(b, b % NSLOT, s).wait()


def kernel(feat, num_unit, v, g, b):
    g2 = jnp.reshape(g, (1,))
    b2 = jnp.reshape(b, (1, D_OUT))
    return pl.pallas_call(
        _body,
        grid=(),
        in_specs=[
            pl.BlockSpec(memory_space=pltpu.SMEM),
            pl.BlockSpec(memory_space=pltpu.SMEM),
            pl.BlockSpec(memory_space=pl.ANY),
            pl.BlockSpec(memory_space=pltpu.VMEM),
            pl.BlockSpec(memory_space=pltpu.VMEM),
        ],
        out_specs=pl.BlockSpec(memory_space=pl.ANY),
        out_shape=jax.ShapeDtypeStruct((B, N, D_OUT), jnp.float32),
        scratch_shapes=[
            pltpu.VMEM((NSLOT, N, D_IN), jnp.float32),
            pltpu.VMEM((NSLOT, N, D_OUT), jnp.float32),
            pltpu.SemaphoreType.DMA((NSLOT, SEG)),
            pltpu.SemaphoreType.DMA((NSLOT, SEG)),
        ],
    )(num_unit, g2, feat, v, b2)


# speedup vs baseline: 3.1205x; 3.1205x over previous
import jax
import jax.numpy as jnp
from jax import lax
from jax.experimental import pallas as pl
from jax.experimental.pallas import tpu as pltpu

B, N, D_IN, D_OUT = 16, 4096, 64, 64


def _body(num_unit_ref, g_ref, feat_ref, v_ref, b_ref, out_ref):
    i = pl.program_id(0)
    nu = num_unit_ref[i]
    x = feat_ref[0]
    lane = lax.broadcasted_iota(jnp.int32, (1, N), 1)
    pen = jnp.where(lane < nu, 0.0, -jnp.inf)
    fmax = jnp.max(x + pen, axis=1, keepdims=True)
    v = v_ref[...]
    norm = jnp.sqrt(jnp.sum(v * v))
    w = v * (g_ref[0] / norm)
    adj = b_ref[...] - lax.dot_general(w, fmax, (((1,), (0,)), ((), ())),
                                       preferred_element_type=jnp.float32)
    out = lax.dot_general(w, x, (((1,), (0,)), ((), ())),
                          preferred_element_type=jnp.float32)
    out_ref[0] = jnp.maximum(out + adj, 0.0)


def kernel(feat, num_unit, v, g, b):
    ft = jnp.transpose(feat, (0, 2, 1))
    g2 = jnp.reshape(g, (1,))
    b2 = jnp.reshape(b, (D_OUT, 1))
    grid_spec = pltpu.PrefetchScalarGridSpec(
        num_scalar_prefetch=2,
        grid=(B,),
        in_specs=[
            pl.BlockSpec((1, D_IN, N), lambda i, *_: (i, 0, 0)),
            pl.BlockSpec((D_OUT, D_IN), lambda i, *_: (0, 0)),
            pl.BlockSpec((D_OUT, 1), lambda i, *_: (0, 0)),
        ],
        out_specs=pl.BlockSpec((1, D_OUT, N), lambda i, *_: (i, 0, 0)),
    )
    out_t = pl.pallas_call(
        _body,
        grid_spec=grid_spec,
        out_shape=jax.ShapeDtypeStruct((B, D_OUT, N), jnp.float32),
    )(num_unit, g2, ft, v, b2)
    return jnp.transpose(out_t, (0, 2, 1))
